# Initial kernel scaffold; baseline (speedup 1.0000x reference)
import jax
import jax.numpy as jnp
from jax.experimental import pallas as pl


def kernel(feats, xyz, W1, b1, g1, be1, W2, b2, g2, be2):
    raise NotImplementedError("write your pallas kernel here")


# speedup vs baseline: 12.3642x; 12.3642x over previous
import functools

import jax
import jax.numpy as jnp
from jax import lax
from jax.experimental import pallas as pl
from jax.experimental.pallas import tpu as pltpu
from jax.experimental.pallas import tpu_sc as plsc

_RADIUS = 0.2
_R2 = _RADIUS * _RADIUS
_NSAMPLE = 32
_EPS = 1e-5

_HIGH = jax.lax.Precision.HIGHEST


def _prep_body(x67_ref, w1_ref, w1x_ref, b1_ref, u_ref, v_ref):
    x = x67_ref[0]
    w1 = w1_ref[...]
    w1x = w1x_ref[...]
    b1 = b1_ref[0]
    u = jnp.einsum("nc,oc->no", x, w1, precision=_HIGH) + b1[None, :]
    v = -jnp.einsum("nc,oc->no", x, w1x, precision=_HIGH)
    n = x.shape[0]
    u_ref[0] = jnp.concatenate([u, jnp.zeros((n, 64), jnp.float32)], axis=1)
    v_ref[0] = v


def _prep(x67, w1, w1x, b1):
    B, N, _ = x67.shape
    return pl.pallas_call(
        _prep_body,
        grid=(B,),
        in_specs=[
            pl.BlockSpec((1, N, 67), lambda b: (b, 0, 0)),
            pl.BlockSpec((64, 67), lambda b: (0, 0)),
            pl.BlockSpec((64, 67), lambda b: (0, 0)),
            pl.BlockSpec((1, 64), lambda b: (0, 0)),
        ],
        out_specs=[
            pl.BlockSpec((1, N, 128), lambda b: (b, 0, 0)),
            pl.BlockSpec((1, N, 64), lambda b: (b, 0, 0)),
        ],
        out_shape=[
            jax.ShapeDtypeStruct((B, N, 128), jnp.float32),
            jax.ShapeDtypeStruct((B, N, 64), jnp.float32),
        ],
    )(x67, w1, w1x, b1)


def _sel_body(q_ref, x_ref, idx_ref):
    b = pl.program_id(0)
    q = q_ref[0]
    x = x_ref[0]
    tm = q.shape[0]
    n = x.shape[0]
    d2 = (jnp.sum(q * q, axis=1)[:, None] + jnp.sum(x * x, axis=1)[None, :]
          - 2.0 * jnp.einsum("md,nd->mn", q, x))
    mask = (d2 < _R2).astype(jnp.float32)
    prefix = mask
    k = 1
    while k < n:
        prefix = prefix + jnp.concatenate(
            [jnp.zeros((tm, k), jnp.float32), prefix[:, :-k]], axis=1)
        k *= 2
    total = prefix[:, n - 1]
    parts = []
    for s in range(_NSAMPLE):
        cnt = jnp.sum((prefix <= jnp.float32(s)).astype(jnp.float32), axis=1)
        parts.append(cnt[:, None])
    idx = jnp.concatenate(parts, axis=1)
    first = jnp.where(total[:, None] > 0, idx[:, 0:1], 0.0)
    svec = lax.broadcasted_iota(jnp.int32, (1, _NSAMPLE), 1).astype(jnp.float32)
    idx = jnp.where(svec < total[:, None], idx, first)
    idx_ref[0] = idx.astype(jnp.int32) + b * n


def _select(xyz8, tm=256):
    B, N, _ = xyz8.shape
    return pl.pallas_call(
        _sel_body,
        grid=(B, N // tm),
        in_specs=[
            pl.BlockSpec((1, tm, 8), lambda b, i: (b, i, 0)),
            pl.BlockSpec((1, N, 8), lambda b, i: (b, 0, 0)),
        ],
        out_specs=pl.BlockSpec((1, tm, _NSAMPLE), lambda b, i: (b, i, 0)),
        out_shape=jax.ShapeDtypeStruct((B, N, _NSAMPLE), jnp.int32),
    )(xyz8, xyz8)


def _scg_body(u_hbm, idx_hbm, g_hbm, idxv, gbuf, sem):
    nrows = g_hbm.shape[0]
    per_w = nrows // 32
    wid = lax.axis_index("s") * 2 + lax.axis_index("c")
    row0 = wid * per_w

    def it(i, carry):
        p = row0 + i * 128
        pltpu.sync_copy(idx_hbm.at[pl.ds(p, 128)], idxv)
        pltpu.async_copy(u_hbm.at[idxv], gbuf, sem).wait()
        pltpu.sync_copy(gbuf, g_hbm.at[pl.ds(p, 128)])
        return carry

    lax.fori_loop(0, per_w // 128, it, jnp.int32(0))


def _sc_gather(u2d, idxflat):
    BN = u2d.shape[0]
    mesh = plsc.VectorSubcoreMesh(core_axis_name="c", subcore_axis_name="s")
    f = pl.kernel(
        _scg_body,
        out_type=jax.ShapeDtypeStruct((BN * _NSAMPLE, 128), jnp.float32),
        mesh=mesh,
        scratch_types=[
            pltpu.VMEM((128,), jnp.int32),
            pltpu.VMEM((128, 128), jnp.float32),
            pltpu.SemaphoreType.DMA,
        ],
    )
    return f(u2d, idxflat)


def _stats1_body(g_ref, v_ref, o_ref):
    pid = pl.program_id(0)

    @pl.when(pid == 0)
    def _():
        o_ref[...] = jnp.zeros_like(o_ref)

    y = g_ref[...][:, :, :64] + v_ref[...][:, None, :]
    o_ref[0, :] += jnp.sum(y, axis=(0, 1))
    o_ref[1, :] += jnp.sum(y * y, axis=(0, 1))


def _stats1(g3, v2d, tm=512):
    R = g3.shape[0]
    return pl.pallas_call(
        _stats1_body,
        grid=(R // tm,),
        in_specs=[
            pl.BlockSpec((tm, _NSAMPLE, 128), lambda i: (i, 0, 0)),
            pl.BlockSpec((tm, 64), lambda i: (i, 0)),
        ],
        out_specs=pl.BlockSpec((2, 64), lambda i: (0, 0)),
        out_shape=jax.ShapeDtypeStruct((2, 64), jnp.float32),
    )(g3, v2d)


def _fused_body(g_ref, v_ref, a1_ref, d1_ref, w2_ref, mx_ref, mn_ref, st_ref):
    pid = pl.program_id(0)

    @pl.when(pid == 0)
    def _():
        st_ref[...] = jnp.zeros_like(st_ref)

    tm = g_ref.shape[0]
    y1 = g_ref[...][:, :, :64] + v_ref[...][:, None, :]
    a1 = a1_ref[0]
    d1 = d1_ref[0]
    h = y1 * a1[None, None, :] + d1[None, None, :]
    h = jnp.where(h >= 0, h, 0.2 * h)
    h2 = h.reshape(tm * _NSAMPLE, 64)
    y2 = jnp.einsum("rk,ok->ro", h2, w2_ref[...], precision=_HIGH)
    st_ref[0, :] += jnp.sum(y2, axis=0)
    st_ref[1, :] += jnp.sum(y2 * y2, axis=0)
    y23 = y2.reshape(tm, _NSAMPLE, 128)
    mx_ref[...] = jnp.max(y23, axis=1)
    mn_ref[...] = jnp.min(y23, axis=1)


def _fused(g3, v2d, a1, d1, w2, tm=256):
    R = g3.shape[0]
    return pl.pallas_call(
        _fused_body,
        grid=(R // tm,),
        in_specs=[
            pl.BlockSpec((tm, _NSAMPLE, 128), lambda i: (i, 0, 0)),
            pl.BlockSpec((tm, 64), lambda i: (i, 0)),
            pl.BlockSpec((1, 64), lambda i: (0, 0)),
            pl.BlockSpec((1, 64), lambda i: (0, 0)),
            pl.BlockSpec((128, 64), lambda i: (0, 0)),
        ],
        out_specs=[
            pl.BlockSpec((tm, 128), lambda i: (i, 0)),
            pl.BlockSpec((tm, 128), lambda i: (i, 0)),
            pl.BlockSpec((2, 128), lambda i: (0, 0)),
        ],
        out_shape=[
            jax.ShapeDtypeStruct((R, 128), jnp.float32),
            jax.ShapeDtypeStruct((R, 128), jnp.float32),
            jax.ShapeDtypeStruct((2, 128), jnp.float32),
        ],
    )(g3, v2d, a1, d1, w2)


def _final_body(mx_ref, mn_ref, a2_ref, d2_ref, o_ref):
    a2 = a2_ref[0]
    d2 = d2_ref[0]
    sel = jnp.where((a2 >= 0)[None, :], mx_ref[...], mn_ref[...])
    o = sel * a2[None, :] + d2[None, :]
    o_ref[...] = jnp.where(o >= 0, o, 0.2 * o)


def _final(mx, mn, a2, d2, tm=1024):
    R = mx.shape[0]
    return pl.pallas_call(
        _final_body,
        grid=(R // tm,),
        in_specs=[
            pl.BlockSpec((tm, 128), lambda i: (i, 0)),
            pl.BlockSpec((tm, 128), lambda i: (i, 0)),
            pl.BlockSpec((1, 128), lambda i: (0, 0)),
            pl.BlockSpec((1, 128), lambda i: (0, 0)),
        ],
        out_specs=pl.BlockSpec((tm, 128), lambda i: (i, 0)),
        out_shape=jax.ShapeDtypeStruct((R, 128), jnp.float32),
    )(mx, mn, a2, d2)


def kernel(feats, xyz, W1, b1, g1, be1, W2, b2, g2, be2):
    B, C, N = feats.shape
    BN = B * N
    NT = jnp.float32(BN * _NSAMPLE)

    feats_nc = jnp.transpose(feats, (0, 2, 1))
    xyz8 = jnp.concatenate([xyz, jnp.zeros((B, N, 5), jnp.float32)], axis=-1)
    x67 = jnp.concatenate([feats_nc, xyz / _RADIUS], axis=-1)
    w1x = jnp.concatenate(
        [jnp.zeros((64, C), jnp.float32), W1[:, C:]], axis=-1)

    u, v = _prep(x67, W1, w1x, b1.reshape(1, 64))
    u2d = u.reshape(BN, 128)
    v2d = v.reshape(BN, 64)

    idx = _select(xyz8)
    idxflat = idx.reshape(BN * _NSAMPLE)
    g = _sc_gather(u2d, idxflat)
    g3 = g.reshape(BN, _NSAMPLE, 128)

    st1 = _stats1(g3, v2d)
    mean1 = st1[0] / NT
    var1 = st1[1] / NT - mean1 * mean1
    a1 = g1 / jnp.sqrt(var1 + _EPS)
    d1 = be1 - a1 * mean1

    mx, mn, st2 = _fused(g3, v2d, a1.reshape(1, 64), d1.reshape(1, 64), W2)
    mean2 = st2[0] / NT
    var2 = st2[1] / NT - mean2 * mean2
    a2 = g2 / jnp.sqrt(var2 + _EPS)
    d2 = be2 - a2 * mean2

    o = _final(mx, mn, a2.reshape(1, 128), d2.reshape(1, 128))
    out_feats = jnp.transpose(o.reshape(B, N, 128), (0, 2, 1))
    return (out_feats, xyz)
